# baseline (speedup 1.0000x reference)
import functools

import jax
import jax.numpy as jnp
from jax import lax
from jax.experimental import pallas as pl
from jax.experimental.pallas import tpu as pltpu
from jax.experimental.pallas import tpu_sc as plsc

N = 10000
DF = 256
H = 8
C = 8
HC = 64
G = 128
NCLS = 3
NEG = 0.2

NP = 10240
NSUB = 16
NCORE = 2
NTILES = NSUB * NCORE
RPS = NP // NSUB
BE = 128
EPT_RAW = -(-(160000 + N) // NTILES)
NB = -(-EPT_RAW // BE)
EP = NTILES * NB * BE
HSW = 80
AW = 72
RB = 2048
F32 = jnp.float32

assert NB % 3 == 0


def _sc_edge_body(hs_hbm, asad_hbm, sd_hbm, z80, acc_out,
                  acc_sh,
                  packed_v, ibs0, ibs1, ibs2, ibd0, ibd1, ibd2,
                  dis0, dis1, dis2, ad0, ad1, ad2, hr0, hr1, hr2,
                  po0, po1, po2, sg0, sg1, sg2, ss0, ss1, ss2):
    cid = lax.axis_index("c")
    sid = lax.axis_index("s")
    wid = cid * NSUB + sid
    r0 = sid * RPS

    pltpu.sync_copy(z80.at[pl.ds(r0, RPS)], acc_sh.at[pl.ds(r0, RPS)])
    pltpu.sync_copy(sd_hbm.at[wid], packed_v)

    iota = lax.iota(jnp.int32, 16)
    row0 = jnp.zeros((16,), jnp.int32)

    plsc.subcore_barrier()

    def unpack(b, ibs, ibd):
        for k in range(BE // 16):
            v = packed_v[pl.ds(b * BE + k * 16, 16)]
            plsc.store_scatter(ibs, [row0, k * 16 + iota], v & 0xFFFF)
            plsc.store_scatter(ibd, [row0, k * 16 + iota],
                               lax.shift_right_logical(v, 16))

    def issue_gathers(ibs, ibd, adr, hr, sg):
        pltpu.async_copy(hs_hbm.at[ibs.at[0]], hr, sg)
        pltpu.async_copy(asad_hbm.at[ibd.at[0]], adr, sg)

    def wait_gathers(ibs, ibd, adr, hr, sg):
        pltpu.make_async_copy(hs_hbm.at[ibs.at[0]], hr, sg).wait()
        pltpu.make_async_copy(asad_hbm.at[ibd.at[0]], adr, sg).wait()

    def wait_scatter(po, dis, ss):
        pltpu.make_async_copy(po, acc_sh.at[dis.at[0]], ss).wait()

    def compute(adr, hr, po):
        def chunk_body(t, _):
            rows = t * 16 + iota
            for hh in range(H):
                colh = jnp.full((16,), hh, jnp.int32)
                av = plsc.load_gather(hr, [rows, colh + HC])
                dv = plsc.load_gather(adr, [rows, colh + 8])
                e = av + dv
                e = jnp.where(e > 0, e, NEG * e)
                exv = jnp.exp(e)
                plsc.store_scatter(po, [rows, colh + HC], exv)
                for cc in range(C):
                    col = jnp.full((16,), hh * C + cc, jnp.int32)
                    hv = plsc.load_gather(hr, [rows, col])
                    plsc.store_scatter(po, [rows, col], hv * exv)
            return 0
        lax.fori_loop(0, BE // 16, chunk_body, 0)

    def snap_dst(ibd, dis):
        for k in range(BE // 16):
            v = plsc.load_gather(ibd, [row0, k * 16 + iota])
            plsc.store_scatter(dis, [row0, k * 16 + iota], v)

    def issue_scatter(po, dis, ss):
        return pltpu.async_copy(po, acc_sh.at[dis.at[0]], ss, add=True)

    sets = ((ibs0, ibd0, dis0, ad0, hr0, po0, sg0, ss0),
            (ibs1, ibd1, dis1, ad1, hr1, po1, sg1, ss1),
            (ibs2, ibd2, dis2, ad2, hr2, po2, sg2, ss2))

    unpack(0, ibs0, ibd0)
    issue_gathers(ibs0, ibd0, ad0, hr0, sg0)
    unpack(1, ibs1, ibd1)
    issue_gathers(ibs1, ibd1, ad1, hr1, sg1)

    def trip_body(i, _):
        for half in range(3):
            b = 3 * i + half
            ibs, ibd, dis, adr, hr, po, sg, ss = sets[half]
            nxt = sets[(half + 2) % 3]

            @pl.when(b + 2 < NB)
            def _():
                unpack(b + 2, nxt[0], nxt[1])
                issue_gathers(nxt[0], nxt[1], nxt[3], nxt[4], nxt[6])

            wait_gathers(ibs, ibd, adr, hr, sg)

            @pl.when(b >= 3)
            def _():
                wait_scatter(po, dis, ss)

            compute(adr, hr, po)
            snap_dst(ibd, dis)
            issue_scatter(po, dis, ss)
        return 0
    lax.fori_loop(0, NB // 3, trip_body, 0)

    wait_scatter(po0, dis0, ss0)
    wait_scatter(po1, dis1, ss1)
    wait_scatter(po2, dis2, ss2)
    plsc.subcore_barrier()
    pltpu.sync_copy(acc_sh.at[pl.ds(r0, RPS)], acc_out.at[cid, pl.ds(r0, RPS)])


@functools.cache
def _sc_edge_kernel():
    return pl.kernel(
        _sc_edge_body,
        out_type=jax.ShapeDtypeStruct((NCORE, NP, AW), F32),
        mesh=plsc.VectorSubcoreMesh(core_axis_name="c", subcore_axis_name="s",
                                    num_cores=NCORE, num_subcores=NSUB),
        compiler_params=pltpu.CompilerParams(needs_layout_passes=False,
                                             use_tc_tiling_on_sc=False),
        scratch_types=(
            [pltpu.VMEM_SHARED((NP, AW), F32),
             pltpu.VMEM((NB * BE,), jnp.int32)]
            + [pltpu.VMEM((1, BE), jnp.int32)] * 9
            + [pltpu.VMEM((BE, 16), F32)] * 3
            + [pltpu.VMEM((BE, HSW), F32)] * 3
            + [pltpu.VMEM((BE, AW), F32)] * 3
            + [pltpu.SemaphoreType.DMA] * 6
        ),
    )


def _sc_edge(*args):
    return _sc_edge_kernel()(*args)


def _l1_body(x_ref, w_ref, a_ref, hs_ref, asad_ref):
    h = jnp.dot(x_ref[...], w_ref[...], preferred_element_type=F32)
    asad = jnp.dot(h, a_ref[...], preferred_element_type=F32)
    hs_ref[...] = jnp.concatenate([h, asad], axis=1)
    asad_ref[...] = asad


_l1_call = pl.pallas_call(
    _l1_body,
    grid=(NP // RB,),
    in_specs=[
        pl.BlockSpec((RB, DF), lambda i: (i, 0)),
        pl.BlockSpec((DF, HC), lambda i: (0, 0)),
        pl.BlockSpec((HC, 16), lambda i: (0, 0)),
    ],
    out_specs=[
        pl.BlockSpec((RB, HSW), lambda i: (i, 0)),
        pl.BlockSpec((RB, 16), lambda i: (i, 0)),
    ],
    out_shape=[
        jax.ShapeDtypeStruct((NP, HSW), F32),
        jax.ShapeDtypeStruct((NP, 16), F32),
    ],
)


def _combine(acc_ref, p_ref, k8_ref):
    o = acc_ref[0, :, 0:HC] + acc_ref[1, :, 0:HC]
    s8 = acc_ref[0, :, HC:HC + 8] + acc_ref[1, :, HC:HC + 8]
    inv = 1.0 / (s8 + 1e-16)
    invex = jnp.dot(inv, k8_ref[...], preferred_element_type=F32)
    agg = o * invex + p_ref[0:1, :]
    eact = jnp.where(agg > 0, agg, jnp.exp(agg) - 1.0)
    return eact * p_ref[1:2, :] + p_ref[2:3, :]


def _mid_body(acc_ref, p_ref, k8_ref, w_ref, a_ref, hs_ref, asad_ref):
    act = _combine(acc_ref, p_ref, k8_ref)
    h = jnp.dot(act, w_ref[...], preferred_element_type=F32)
    asad = jnp.dot(h, a_ref[...], preferred_element_type=F32)
    hs_ref[...] = jnp.concatenate([h, asad], axis=1)
    asad_ref[...] = asad


_mid_call = pl.pallas_call(
    _mid_body,
    grid=(NP // RB,),
    in_specs=[
        pl.BlockSpec((NCORE, RB, AW), lambda i: (0, i, 0)),
        pl.BlockSpec((8, HC), lambda i: (0, 0)),
        pl.BlockSpec((8, HC), lambda i: (0, 0)),
        pl.BlockSpec((HC, HC), lambda i: (0, 0)),
        pl.BlockSpec((HC, 16), lambda i: (0, 0)),
    ],
    out_specs=[
        pl.BlockSpec((RB, HSW), lambda i: (i, 0)),
        pl.BlockSpec((RB, 16), lambda i: (i, 0)),
    ],
    out_shape=[
        jax.ShapeDtypeStruct((NP, HSW), F32),
        jax.ShapeDtypeStruct((NP, 16), F32),
    ],
)


def _pool_body(acc_ref, p_ref, k8_ref, pm_ref, out_ref):
    act = _combine(acc_ref, p_ref, k8_ref)

    @pl.when(pl.program_id(0) == 0)
    def _():
        out_ref[...] = jnp.zeros_like(out_ref)

    out_ref[...] += jnp.dot(pm_ref[...], act, preferred_element_type=F32)


_pool_call = pl.pallas_call(
    _pool_body,
    grid=(NP // RB,),
    in_specs=[
        pl.BlockSpec((NCORE, RB, AW), lambda i: (0, i, 0)),
        pl.BlockSpec((8, HC), lambda i: (0, 0)),
        pl.BlockSpec((8, HC), lambda i: (0, 0)),
        pl.BlockSpec((G, RB), lambda i: (0, i)),
    ],
    out_specs=pl.BlockSpec((G, HC), lambda i: (0, 0)),
    out_shape=jax.ShapeDtypeStruct((G, HC), F32),
)


def _head_body(pooled_ref, w1_ref, b1_ref, w2_ref, b2_ref, out_ref):
    z = jnp.dot(pooled_ref[...], w1_ref[...], preferred_element_type=F32)
    z = jnp.maximum(z + b1_ref[0:1, :], 0.0)
    z = jnp.dot(z, w2_ref[...], preferred_element_type=F32) + b2_ref[0:1, :]
    m = jnp.max(z, axis=-1, keepdims=True)
    lse = m + jnp.log(jnp.sum(jnp.exp(z - m), axis=-1, keepdims=True))
    out_ref[...] = z - lse


_head_call = pl.pallas_call(
    _head_body,
    out_shape=jax.ShapeDtypeStruct((G, 8), F32),
)


def _attn_mat(a_src, a_dst):
    eye = jnp.repeat(jnp.eye(H, dtype=F32), C, axis=0)
    return jnp.concatenate(
        [eye * a_src.reshape(HC, 1), eye * a_dst.reshape(HC, 1)], axis=1)


def kernel(x, edge_index, batch, params):
    xp = jnp.zeros((NP, DF), F32).at[:N].set(x)
    loop = jnp.arange(N, dtype=jnp.int32)
    src = jnp.concatenate([edge_index[0].astype(jnp.int32), loop])
    dst = jnp.concatenate([edge_index[1].astype(jnp.int32), loop])
    pad = jnp.full((EP - src.shape[0],), N, jnp.int32)
    packed = (jnp.concatenate([src, pad])
              | (jnp.concatenate([dst, pad]) << 16)).reshape(NTILES, NB * BE)

    k8 = jnp.repeat(jnp.eye(H, dtype=F32), C, axis=0).T
    z80 = jnp.zeros((NP, AW), F32)
    pmat = jnp.zeros((G, NP), F32).at[:, :N].set(
        (batch[None, :] == jnp.arange(G, dtype=batch.dtype)[:, None])
        .astype(F32))

    bn_scale = 1.0 / jnp.sqrt(jnp.float32(1.0 + 1e-5))
    pvecs = {}
    amats = {}
    for i in (1, 2, 3, 4):
        pv = jnp.zeros((8, HC), F32)
        pv = pv.at[0].set(params['bias%d' % i])
        pv = pv.at[1].set(params['bn_g%d' % i] * bn_scale)
        pv = pv.at[2].set(params['bn_b%d' % i])
        pvecs[i] = pv
        amats[i] = _attn_mat(params['a_src%d' % i], params['a_dst%d' % i])

    b1 = jnp.zeros((8, 32), F32).at[0].set(params['fc1_b'])
    w2 = jnp.zeros((32, 8), F32).at[:, :NCLS].set(params['fc2_W'])
    b2 = jnp.full((8, 8), -1e30, F32).at[0, :NCLS].set(params['fc2_b'])

    hs, asad = _l1_call(xp, params['W1'], amats[1])
    acc = _sc_edge(hs, asad, packed, z80)

    for i in (2, 3, 4):
        hs, asad = _mid_call(acc, pvecs[i - 1], k8,
                             params['W%d' % i], amats[i])
        acc = _sc_edge(hs, asad, packed, z80)

    pooled = _pool_call(acc, pvecs[4], k8, pmat)
    out8 = _head_call(pooled, params['fc1_W'], b1, w2, b2)
    return out8[:, :NCLS]


# speedup vs baseline: 1.0071x; 1.0071x over previous
import functools

import jax
import jax.numpy as jnp
from jax import lax
from jax.experimental import pallas as pl
from jax.experimental.pallas import tpu as pltpu
from jax.experimental.pallas import tpu_sc as plsc

N = 10000
DF = 256
H = 8
C = 8
HC = 64
G = 128
NCLS = 3
NEG = 0.2

NP = 10240
NSUB = 16
NCORE = 2
NTILES = NSUB * NCORE
RPS = NP // NSUB
BE = 128
EPT_RAW = -(-(160000 + N) // NTILES)
NB = -(-EPT_RAW // BE)
EP = NTILES * NB * BE
HSW = 80
AW = 72
RB = 2048
F32 = jnp.float32

assert NB % 3 == 0


def _sc_edge_body(hs_hbm, asad_hbm, sd_hbm, z80, acc_out,
                  acc_sh,
                  packed_v, ibs0, ibs1, ibs2, ibd0, ibd1, ibd2,
                  dis0, dis1, dis2, ad0, ad1, ad2, hr0, hr1, hr2,
                  po0, po1, po2, sg0, sg1, sg2, ss0, ss1, ss2):
    cid = lax.axis_index("c")
    sid = lax.axis_index("s")
    wid = cid * NSUB + sid
    r0 = sid * RPS

    pltpu.sync_copy(z80.at[pl.ds(r0, RPS)], acc_sh.at[pl.ds(r0, RPS)])
    pltpu.sync_copy(sd_hbm.at[wid], packed_v)

    iota = lax.iota(jnp.int32, 16)
    row0 = jnp.zeros((16,), jnp.int32)

    plsc.subcore_barrier()

    def unpack(b, ibs, ibd):
        for k in range(BE // 16):
            v = packed_v[pl.ds(b * BE + k * 16, 16)]
            plsc.store_scatter(ibs, [row0, k * 16 + iota], v & 0xFFFF)
            plsc.store_scatter(ibd, [row0, k * 16 + iota],
                               lax.shift_right_logical(v, 16))

    def issue_gathers(ibs, ibd, adr, hr, sg):
        pltpu.async_copy(hs_hbm.at[ibs.at[0]], hr, sg)
        pltpu.async_copy(asad_hbm.at[ibd.at[0]], adr, sg)

    def wait_gathers(ibs, ibd, adr, hr, sg):
        pltpu.make_async_copy(hs_hbm.at[ibs.at[0]], hr, sg).wait()
        pltpu.make_async_copy(asad_hbm.at[ibd.at[0]], adr, sg).wait()

    def wait_scatter(po, dis, ss):
        pltpu.make_async_copy(po, acc_sh.at[dis.at[0]], ss).wait()

    def compute(adr, hr, po):
        def chunk_body(t, _):
            rows = t * 16 + iota
            for hh in range(H):
                colh = jnp.full((16,), hh, jnp.int32)
                av = plsc.load_gather(hr, [rows, colh + HC])
                dv = plsc.load_gather(adr, [rows, colh + 8])
                e = av + dv
                e = jnp.where(e > 0, e, NEG * e)
                exv = jnp.exp(e)
                plsc.store_scatter(po, [rows, colh + HC], exv)
                for cc in range(C):
                    col = jnp.full((16,), hh * C + cc, jnp.int32)
                    hv = plsc.load_gather(hr, [rows, col])
                    plsc.store_scatter(po, [rows, col], hv * exv)
            return 0
        lax.fori_loop(0, BE // 16, chunk_body, 0)

    def snap_dst(ibd, dis):
        for k in range(BE // 16):
            v = plsc.load_gather(ibd, [row0, k * 16 + iota])
            plsc.store_scatter(dis, [row0, k * 16 + iota], v)

    def issue_scatter(po, dis, ss):
        return pltpu.async_copy(po, acc_sh.at[dis.at[0]], ss, add=True)

    sets = ((ibs0, ibd0, dis0, ad0, hr0, po0, sg0, ss0),
            (ibs1, ibd1, dis1, ad1, hr1, po1, sg1, ss1),
            (ibs2, ibd2, dis2, ad2, hr2, po2, sg2, ss2))

    unpack(0, ibs0, ibd0)
    issue_gathers(ibs0, ibd0, ad0, hr0, sg0)
    unpack(1, ibs1, ibd1)
    issue_gathers(ibs1, ibd1, ad1, hr1, sg1)

    def trip_body(i, _):
        for half in range(3):
            b = 3 * i + half
            ibs, ibd, dis, adr, hr, po, sg, ss = sets[half]
            nxt = sets[(half + 2) % 3]

            @pl.when(b + 2 < NB)
            def _():
                unpack(b + 2, nxt[0], nxt[1])
                issue_gathers(nxt[0], nxt[1], nxt[3], nxt[4], nxt[6])

            wait_gathers(ibs, ibd, adr, hr, sg)

            @pl.when(b >= 3)
            def _():
                wait_scatter(po, dis, ss)

            compute(adr, hr, po)
            snap_dst(ibd, dis)
            issue_scatter(po, dis, ss)
        return 0
    lax.fori_loop(0, NB // 3, trip_body, 0)

    wait_scatter(po0, dis0, ss0)
    wait_scatter(po1, dis1, ss1)
    wait_scatter(po2, dis2, ss2)
    plsc.subcore_barrier()
    pltpu.sync_copy(acc_sh.at[pl.ds(r0, RPS)], acc_out.at[cid, pl.ds(r0, RPS)])


@functools.cache
def _sc_edge_kernel():
    return pl.kernel(
        _sc_edge_body,
        out_type=jax.ShapeDtypeStruct((NCORE, NP, AW), F32),
        mesh=plsc.VectorSubcoreMesh(core_axis_name="c", subcore_axis_name="s",
                                    num_cores=NCORE, num_subcores=NSUB),
        compiler_params=pltpu.CompilerParams(needs_layout_passes=False,
                                             use_tc_tiling_on_sc=False),
        scratch_types=(
            [pltpu.VMEM_SHARED((NP, AW), F32),
             pltpu.VMEM((NB * BE,), jnp.int32)]
            + [pltpu.VMEM((1, BE), jnp.int32)] * 9
            + [pltpu.VMEM((BE, 16), F32)] * 3
            + [pltpu.VMEM((BE, HSW), F32)] * 3
            + [pltpu.VMEM((BE, AW), F32)] * 3
            + [pltpu.SemaphoreType.DMA] * 6
        ),
    )


def _sc_edge(*args):
    return _sc_edge_kernel()(*args)


def _l1_body(x_ref, w_ref, a_ref, hs_ref, asad_ref):
    h = jnp.dot(x_ref[...], w_ref[...], preferred_element_type=F32)
    asad = jnp.dot(h, a_ref[...], preferred_element_type=F32)
    hs_ref[...] = jnp.concatenate([h, asad], axis=1)
    asad_ref[...] = asad


_l1_call = pl.pallas_call(
    _l1_body,
    grid=(NP // RB,),
    in_specs=[
        pl.BlockSpec((RB, DF), lambda i: (i, 0)),
        pl.BlockSpec((DF, HC), lambda i: (0, 0)),
        pl.BlockSpec((HC, 16), lambda i: (0, 0)),
    ],
    out_specs=[
        pl.BlockSpec((RB, HSW), lambda i: (i, 0)),
        pl.BlockSpec((RB, 16), lambda i: (i, 0)),
    ],
    out_shape=[
        jax.ShapeDtypeStruct((NP, HSW), F32),
        jax.ShapeDtypeStruct((NP, 16), F32),
    ],
)


def _combine(acc_ref, p_ref, k8_ref):
    o = acc_ref[0, :, 0:HC] + acc_ref[1, :, 0:HC]
    s8 = acc_ref[0, :, HC:HC + 8] + acc_ref[1, :, HC:HC + 8]
    inv = 1.0 / (s8 + 1e-16)
    invex = jnp.dot(inv, k8_ref[...], preferred_element_type=F32)
    agg = o * invex + p_ref[0:1, :]
    eact = jnp.where(agg > 0, agg, jnp.exp(agg) - 1.0)
    return eact * p_ref[1:2, :] + p_ref[2:3, :]


def _mid_body(acc_ref, p_ref, k8_ref, w_ref, a_ref, hs_ref, asad_ref):
    act = _combine(acc_ref, p_ref, k8_ref)
    h = jnp.dot(act, w_ref[...], preferred_element_type=F32)
    asad = jnp.dot(h, a_ref[...], preferred_element_type=F32)
    hs_ref[...] = jnp.concatenate([h, asad], axis=1)
    asad_ref[...] = asad


_mid_call = pl.pallas_call(
    _mid_body,
    grid=(NP // RB,),
    in_specs=[
        pl.BlockSpec((NCORE, RB, AW), lambda i: (0, i, 0)),
        pl.BlockSpec((8, HC), lambda i: (0, 0)),
        pl.BlockSpec((8, HC), lambda i: (0, 0)),
        pl.BlockSpec((HC, HC), lambda i: (0, 0)),
        pl.BlockSpec((HC, 16), lambda i: (0, 0)),
    ],
    out_specs=[
        pl.BlockSpec((RB, HSW), lambda i: (i, 0)),
        pl.BlockSpec((RB, 16), lambda i: (i, 0)),
    ],
    out_shape=[
        jax.ShapeDtypeStruct((NP, HSW), F32),
        jax.ShapeDtypeStruct((NP, 16), F32),
    ],
)


def _pool_body(acc_ref, p_ref, k8_ref, pm_ref, w1_ref, b1_ref, w2_ref,
               b2_ref, pooled_ref, out_ref):
    act = _combine(acc_ref, p_ref, k8_ref)

    @pl.when(pl.program_id(0) == 0)
    def _():
        pooled_ref[...] = jnp.zeros_like(pooled_ref)

    pooled_ref[...] += jnp.dot(pm_ref[...], act, preferred_element_type=F32)

    @pl.when(pl.program_id(0) == NP // RB - 1)
    def _():
        z = jnp.dot(pooled_ref[...], w1_ref[...], preferred_element_type=F32)
        z = jnp.maximum(z + b1_ref[0:1, :], 0.0)
        z = (jnp.dot(z, w2_ref[...], preferred_element_type=F32)
             + b2_ref[0:1, :])
        m = jnp.max(z, axis=-1, keepdims=True)
        lse = m + jnp.log(jnp.sum(jnp.exp(z - m), axis=-1, keepdims=True))
        out_ref[...] = z - lse


_pool_call = pl.pallas_call(
    _pool_body,
    grid=(NP // RB,),
    in_specs=[
        pl.BlockSpec((NCORE, RB, AW), lambda i: (0, i, 0)),
        pl.BlockSpec((8, HC), lambda i: (0, 0)),
        pl.BlockSpec((8, HC), lambda i: (0, 0)),
        pl.BlockSpec((G, RB), lambda i: (0, i)),
        pl.BlockSpec((HC, 32), lambda i: (0, 0)),
        pl.BlockSpec((8, 32), lambda i: (0, 0)),
        pl.BlockSpec((32, 8), lambda i: (0, 0)),
        pl.BlockSpec((8, 8), lambda i: (0, 0)),
    ],
    out_specs=[
        pl.BlockSpec((G, HC), lambda i: (0, 0)),
        pl.BlockSpec((G, 8), lambda i: (0, 0)),
    ],
    out_shape=[
        jax.ShapeDtypeStruct((G, HC), F32),
        jax.ShapeDtypeStruct((G, 8), F32),
    ],
)


def _attn_mat(a_src, a_dst):
    eye = jnp.repeat(jnp.eye(H, dtype=F32), C, axis=0)
    return jnp.concatenate(
        [eye * a_src.reshape(HC, 1), eye * a_dst.reshape(HC, 1)], axis=1)


def kernel(x, edge_index, batch, params):
    xp = jnp.zeros((NP, DF), F32).at[:N].set(x)
    loop = jnp.arange(N, dtype=jnp.int32)
    src = jnp.concatenate([edge_index[0].astype(jnp.int32), loop])
    dst = jnp.concatenate([edge_index[1].astype(jnp.int32), loop])
    pad = jnp.full((EP - src.shape[0],), N, jnp.int32)
    packed = (jnp.concatenate([src, pad])
              | (jnp.concatenate([dst, pad]) << 16)).reshape(NTILES, NB * BE)

    k8 = jnp.repeat(jnp.eye(H, dtype=F32), C, axis=0).T
    z80 = jnp.zeros((NP, AW), F32)
    pmat = jnp.zeros((G, NP), F32).at[:, :N].set(
        (batch[None, :] == jnp.arange(G, dtype=batch.dtype)[:, None])
        .astype(F32))

    bn_scale = 1.0 / jnp.sqrt(jnp.float32(1.0 + 1e-5))
    pvecs = {}
    amats = {}
    for i in (1, 2, 3, 4):
        pv = jnp.zeros((8, HC), F32)
        pv = pv.at[0].set(params['bias%d' % i])
        pv = pv.at[1].set(params['bn_g%d' % i] * bn_scale)
        pv = pv.at[2].set(params['bn_b%d' % i])
        pvecs[i] = pv
        amats[i] = _attn_mat(params['a_src%d' % i], params['a_dst%d' % i])

    b1 = jnp.zeros((8, 32), F32).at[0].set(params['fc1_b'])
    w2 = jnp.zeros((32, 8), F32).at[:, :NCLS].set(params['fc2_W'])
    b2 = jnp.full((8, 8), -1e30, F32).at[0, :NCLS].set(params['fc2_b'])

    hs, asad = _l1_call(xp, params['W1'], amats[1])
    acc = _sc_edge(hs, asad, packed, z80)

    for i in (2, 3, 4):
        hs, asad = _mid_call(acc, pvecs[i - 1], k8,
                             params['W%d' % i], amats[i])
        acc = _sc_edge(hs, asad, packed, z80)

    _, out8 = _pool_call(acc, pvecs[4], k8, pmat, params['fc1_W'], b1, w2, b2)
    return out8[:, :NCLS]
